# baseline (speedup 1.0000x reference)
import functools

import jax
import jax.numpy as jnp
from jax.experimental import pallas as pl
from jax.experimental.pallas import tpu as pltpu

_POOL = 7
_SR = 2
_SCALE = 0.0625
_H = 64
_W = 64


def _roi_kernel(rois_ref, f_ref, o_ref):
    n = pl.program_id(1)
    b = rois_ref[n, 0].astype(jnp.int32)
    rsw = rois_ref[n, 1] * _SCALE
    rsh = rois_ref[n, 2] * _SCALE
    rew = rois_ref[n, 3] * _SCALE
    reh = rois_ref[n, 4] * _SCALE
    roi_w = jnp.maximum(rew - rsw, 1.0)
    roi_h = jnp.maximum(reh - rsh, 1.0)
    bin_h = roi_h / _POOL
    bin_w = roi_w / _POOL

    def interp(v, size):
        valid = (v >= -1.0) & (v <= size)
        vc = jnp.maximum(v, 0.0)
        lo = jnp.minimum(jnp.floor(vc).astype(jnp.int32), size - 1)
        hi = jnp.minimum(lo + 1, size - 1)
        vv = jnp.where(lo >= size - 1, jnp.float32(size - 1), vc)
        l = vv - lo.astype(jnp.float32)
        m = valid.astype(jnp.float32)
        return lo, hi, (1.0 - l) * m, l * m

    iota_j = jax.lax.broadcasted_iota(jnp.int32, (_POOL, _W), 0)
    iota_w = jax.lax.broadcasted_iota(jnp.int32, (_POOL, _W), 1)
    xw = jnp.zeros((_POOL, _W), dtype=jnp.float32)
    for j in range(_POOL):
        for t in range(_SR):
            x = rsw + (j + (t + 0.5) / _SR) * bin_w
            lo, hi, wlo, whi = interp(x, _W)
            sel_j = iota_j == j
            xw = xw + jnp.where(sel_j & (iota_w == lo), wlo * 0.25, 0.0)
            xw = xw + jnp.where(sel_j & (iota_w == hi), whi * 0.25, 0.0)
    xwt = jnp.tile(xw.T, (1, _POOL)).astype(jnp.bfloat16)
    lane_bin = jax.lax.broadcasted_iota(jnp.int32, (_W, _POOL * _POOL), 1) // _POOL

    res = None
    for i in range(_POOL):
        acc = None
        for s in range(_SR):
            y = rsh + (i + (s + 0.5) / _SR) * bin_h
            lo, hi, wlo, whi = interp(y, _H)
            r_lo = f_ref[b, lo, :, :]
            r_hi = f_ref[b, hi, :, :]
            term = (wlo.astype(jnp.bfloat16) * r_lo
                    + whi.astype(jnp.bfloat16) * r_hi)
            acc = term if acc is None else acc + term
        xw_i = jnp.where(lane_bin == i, xwt, jnp.bfloat16(0.0))
        part = jax.lax.dot_general(
            acc, xw_i, (((0,), (0,)), ((), ())),
            preferred_element_type=jnp.float32)
        res = part if res is None else res + part
    o_ref[0] = res


def _run(ft, rois, cb):
    n_rois = rois.shape[0]
    cl = ft.shape[3]
    nc = cl // cb
    grid = (nc, n_rois)
    return pl.pallas_call(
        _roi_kernel,
        grid=grid,
        in_specs=[
            pl.BlockSpec(memory_space=pltpu.SMEM),
            pl.BlockSpec((2, _H, _W, cb), lambda c, n: (0, 0, 0, c)),
        ],
        out_specs=pl.BlockSpec((1, cb, _POOL * _POOL), lambda c, n: (n, c, 0)),
        out_shape=jax.ShapeDtypeStruct((n_rois, cl, _POOL * _POOL), jnp.float32),
    )(rois, ft)


@jax.jit
def kernel(input, rois):
    B, C, L, H, W = input.shape
    ft = input.astype(jnp.bfloat16).transpose(0, 3, 4, 1, 2).reshape(
        B, H, W, C * L)
    out = _run(ft, rois, 2048)
    return out.reshape(rois.shape[0], C, L, _POOL, _POOL)


# speedup vs baseline: 1.6330x; 1.6330x over previous
import functools

import jax
import jax.numpy as jnp
from jax.experimental import pallas as pl
from jax.experimental.pallas import tpu as pltpu

_POOL = 7
_SR = 2
_SCALE = 0.0625
_H = 64
_W = 64


def _roi_kernel(rois_ref, f_ref, o_ref):
    n = pl.program_id(1)
    b = rois_ref[n, 0].astype(jnp.int32)
    rsw = rois_ref[n, 1] * _SCALE
    rsh = rois_ref[n, 2] * _SCALE
    rew = rois_ref[n, 3] * _SCALE
    reh = rois_ref[n, 4] * _SCALE
    roi_w = jnp.maximum(rew - rsw, 1.0)
    roi_h = jnp.maximum(reh - rsh, 1.0)
    bin_h = roi_h / _POOL
    bin_w = roi_w / _POOL

    def interp(v, size):
        valid = (v >= -1.0) & (v <= size)
        vc = jnp.maximum(v, 0.0)
        lo = jnp.minimum(jnp.floor(vc).astype(jnp.int32), size - 1)
        hi = jnp.minimum(lo + 1, size - 1)
        vv = jnp.where(lo >= size - 1, jnp.float32(size - 1), vc)
        l = vv - lo.astype(jnp.float32)
        m = valid.astype(jnp.float32)
        return lo, hi, (1.0 - l) * m, l * m

    iota_j = jax.lax.broadcasted_iota(jnp.int32, (_POOL, _W), 0)
    iota_w = jax.lax.broadcasted_iota(jnp.int32, (_POOL, _W), 1)
    xw = jnp.zeros((_POOL, _W), dtype=jnp.float32)
    for j in range(_POOL):
        for t in range(_SR):
            x = rsw + (j + (t + 0.5) / _SR) * bin_w
            lo, hi, wlo, whi = interp(x, _W)
            sel_j = iota_j == j
            xw = xw + jnp.where(sel_j & (iota_w == lo), wlo * 0.25, 0.0)
            xw = xw + jnp.where(sel_j & (iota_w == hi), whi * 0.25, 0.0)
    xw = xw.astype(jnp.bfloat16)

    for i in range(_POOL):
        acc = None
        for s in range(_SR):
            y = rsh + (i + (s + 0.5) / _SR) * bin_h
            lo, hi, wlo, whi = interp(y, _H)
            r_lo = f_ref[b, lo, :, :]
            r_hi = f_ref[b, hi, :, :]
            term = (wlo.astype(jnp.bfloat16) * r_lo
                    + whi.astype(jnp.bfloat16) * r_hi)
            acc = term if acc is None else acc + term
        res = jax.lax.dot_general(
            xw, acc, (((1,), (0,)), ((), ())),
            preferred_element_type=jnp.float32)
        o_ref[0, i] = res


def _run(ft, rois, cb):
    n_rois = rois.shape[0]
    cl = ft.shape[3]
    nc = cl // cb
    grid = (nc, n_rois)
    return pl.pallas_call(
        _roi_kernel,
        grid=grid,
        in_specs=[
            pl.BlockSpec(memory_space=pltpu.SMEM),
            pl.BlockSpec((2, _H, _W, cb), lambda c, n: (0, 0, 0, c)),
        ],
        out_specs=pl.BlockSpec((1, _POOL, _POOL, cb), lambda c, n: (n, 0, 0, c)),
        out_shape=jax.ShapeDtypeStruct((n_rois, _POOL, _POOL, cl), jnp.float32),
    )(rois, ft)


@jax.jit
def kernel(input, rois):
    B, C, L, H, W = input.shape
    ft = input.astype(jnp.bfloat16).transpose(0, 3, 4, 1, 2).reshape(
        B, H, W, C * L)
    out = _run(ft, rois, 2048)
    return out.transpose(0, 3, 1, 2).reshape(rois.shape[0], C, L, _POOL, _POOL)


# speedup vs baseline: 1.7433x; 1.0676x over previous
import functools

import jax
import jax.numpy as jnp
from jax.experimental import pallas as pl
from jax.experimental.pallas import tpu as pltpu

_POOL = 7
_SR = 2
_SCALE = 0.0625
_H = 64
_W = 64
_NR = 2


def _one_roi(rois_ref, f_ref, o_ref, n, k):
    b = rois_ref[n, 0].astype(jnp.int32)
    rsw = rois_ref[n, 1] * _SCALE
    rsh = rois_ref[n, 2] * _SCALE
    rew = rois_ref[n, 3] * _SCALE
    reh = rois_ref[n, 4] * _SCALE
    roi_w = jnp.maximum(rew - rsw, 1.0)
    roi_h = jnp.maximum(reh - rsh, 1.0)
    bin_h = roi_h / _POOL
    bin_w = roi_w / _POOL

    def interp(v, size):
        valid = (v >= -1.0) & (v <= size)
        vc = jnp.maximum(v, 0.0)
        lo = jnp.minimum(jnp.floor(vc).astype(jnp.int32), size - 1)
        hi = jnp.minimum(lo + 1, size - 1)
        vv = jnp.where(lo >= size - 1, jnp.float32(size - 1), vc)
        l = vv - lo.astype(jnp.float32)
        m = valid.astype(jnp.float32)
        return lo, hi, (1.0 - l) * m, l * m

    iota_j = jax.lax.broadcasted_iota(jnp.int32, (_POOL, _W), 0)
    iota_w = jax.lax.broadcasted_iota(jnp.int32, (_POOL, _W), 1)
    xw = jnp.zeros((_POOL, _W), dtype=jnp.float32)
    for j in range(_POOL):
        for t in range(_SR):
            x = rsw + (j + (t + 0.5) / _SR) * bin_w
            lo, hi, wlo, whi = interp(x, _W)
            sel_j = iota_j == j
            xw = xw + jnp.where(sel_j & (iota_w == lo), wlo * 0.25, 0.0)
            xw = xw + jnp.where(sel_j & (iota_w == hi), whi * 0.25, 0.0)
    xw = xw.astype(jnp.bfloat16)

    for i in range(_POOL):
        acc = None
        for s in range(_SR):
            y = rsh + (i + (s + 0.5) / _SR) * bin_h
            lo, hi, wlo, whi = interp(y, _H)
            r_lo = f_ref[b, lo, :, :]
            r_hi = f_ref[b, hi, :, :]
            term = (wlo.astype(jnp.bfloat16) * r_lo
                    + whi.astype(jnp.bfloat16) * r_hi)
            acc = term if acc is None else acc + term
        res = jax.lax.dot_general(
            xw, acc, (((1,), (0,)), ((), ())),
            preferred_element_type=jnp.float32)
        o_ref[k, i] = res


def _roi_kernel(rois_ref, f_ref, o_ref):
    n0 = pl.program_id(1) * _NR
    for k in range(_NR):
        _one_roi(rois_ref, f_ref, o_ref, n0 + k, k)


def _run(ft, rois, cb):
    n_rois = rois.shape[0]
    cl = ft.shape[3]
    nc = cl // cb
    grid = (nc, n_rois // _NR)
    return pl.pallas_call(
        _roi_kernel,
        grid=grid,
        in_specs=[
            pl.BlockSpec(memory_space=pltpu.SMEM),
            pl.BlockSpec((2, _H, _W, cb), lambda c, n: (0, 0, 0, c)),
        ],
        out_specs=pl.BlockSpec(
            (_NR, _POOL, _POOL, cb), lambda c, n: (n, 0, 0, c)),
        out_shape=jax.ShapeDtypeStruct((n_rois, _POOL, _POOL, cl), jnp.float32),
    )(rois, ft)


@jax.jit
def kernel(input, rois):
    B, C, L, H, W = input.shape
    ft = input.astype(jnp.bfloat16).transpose(0, 3, 4, 1, 2).reshape(
        B, H, W, C * L)
    out = _run(ft, rois, 2048)
    return out.transpose(0, 3, 1, 2).reshape(rois.shape[0], C, L, _POOL, _POOL)


# speedup vs baseline: 1.8167x; 1.0421x over previous
import functools

import jax
import jax.numpy as jnp
from jax.experimental import pallas as pl
from jax.experimental.pallas import tpu as pltpu

_POOL = 7
_SR = 2
_SCALE = 0.0625
_H = 64
_W = 64
_NR = 4


def _one_roi(rois_ref, f_ref, o_ref, n, k):
    b = rois_ref[n, 0].astype(jnp.int32)
    rsw = rois_ref[n, 1] * _SCALE
    rsh = rois_ref[n, 2] * _SCALE
    rew = rois_ref[n, 3] * _SCALE
    reh = rois_ref[n, 4] * _SCALE
    roi_w = jnp.maximum(rew - rsw, 1.0)
    roi_h = jnp.maximum(reh - rsh, 1.0)
    bin_h = roi_h / _POOL
    bin_w = roi_w / _POOL

    def interp(v, size):
        valid = (v >= -1.0) & (v <= size)
        vc = jnp.maximum(v, 0.0)
        lo = jnp.minimum(jnp.floor(vc).astype(jnp.int32), size - 1)
        hi = jnp.minimum(lo + 1, size - 1)
        vv = jnp.where(lo >= size - 1, jnp.float32(size - 1), vc)
        l = vv - lo.astype(jnp.float32)
        m = valid.astype(jnp.float32)
        return lo, hi, (1.0 - l) * m, l * m

    iota_j = jax.lax.broadcasted_iota(jnp.int32, (_POOL, _W), 0)
    iota_w = jax.lax.broadcasted_iota(jnp.int32, (_POOL, _W), 1)
    xw = jnp.zeros((_POOL, _W), dtype=jnp.float32)
    for j in range(_POOL):
        for t in range(_SR):
            x = rsw + (j + (t + 0.5) / _SR) * bin_w
            lo, hi, wlo, whi = interp(x, _W)
            sel_j = iota_j == j
            xw = xw + jnp.where(sel_j & (iota_w == lo), wlo * 0.25, 0.0)
            xw = xw + jnp.where(sel_j & (iota_w == hi), whi * 0.25, 0.0)
    xw = xw.astype(jnp.bfloat16)

    for i in range(_POOL):
        acc = None
        for s in range(_SR):
            y = rsh + (i + (s + 0.5) / _SR) * bin_h
            lo, hi, wlo, whi = interp(y, _H)
            r_lo = f_ref[b, lo, :, :]
            r_hi = f_ref[b, hi, :, :]
            term = (wlo.astype(jnp.bfloat16) * r_lo
                    + whi.astype(jnp.bfloat16) * r_hi)
            acc = term if acc is None else acc + term
        res = jax.lax.dot_general(
            xw, acc, (((1,), (0,)), ((), ())),
            preferred_element_type=jnp.float32)
        o_ref[k, i] = res


def _roi_kernel(rois_ref, f_ref, o_ref):
    n0 = pl.program_id(1) * _NR
    for k in range(_NR):
        _one_roi(rois_ref, f_ref, o_ref, n0 + k, k)


def _run(ft, rois, cb):
    n_rois = rois.shape[0]
    cl = ft.shape[3]
    nc = cl // cb
    grid = (nc, n_rois // _NR)
    return pl.pallas_call(
        _roi_kernel,
        grid=grid,
        in_specs=[
            pl.BlockSpec(memory_space=pltpu.SMEM),
            pl.BlockSpec((2, _H, _W, cb), lambda c, n: (0, 0, 0, c)),
        ],
        out_specs=pl.BlockSpec(
            (_NR, _POOL, _POOL, cb), lambda c, n: (n, 0, 0, c)),
        out_shape=jax.ShapeDtypeStruct((n_rois, _POOL, _POOL, cl), jnp.float32),
    )(rois, ft)


@jax.jit
def kernel(input, rois):
    B, C, L, H, W = input.shape
    ft = input.astype(jnp.bfloat16).transpose(0, 3, 4, 1, 2).reshape(
        B, H, W, C * L)
    out = _run(ft, rois, 2048)
    return out.transpose(0, 3, 1, 2).reshape(rois.shape[0], C, L, _POOL, _POOL)


# speedup vs baseline: 1.9441x; 1.0702x over previous
import functools

import jax
import jax.numpy as jnp
from jax.experimental import pallas as pl
from jax.experimental.pallas import tpu as pltpu

_POOL = 7
_SR = 2
_SCALE = 0.0625
_H = 64
_W = 64
_NR = 4


def _one_roi(rois_ref, f_ref, o_ref, n, k):
    b = rois_ref[n, 0].astype(jnp.int32)
    rsw = rois_ref[n, 1] * _SCALE
    rsh = rois_ref[n, 2] * _SCALE
    rew = rois_ref[n, 3] * _SCALE
    reh = rois_ref[n, 4] * _SCALE
    roi_w = jnp.maximum(rew - rsw, 1.0)
    roi_h = jnp.maximum(reh - rsh, 1.0)
    bin_h = roi_h / _POOL
    bin_w = roi_w / _POOL

    def interp(v, size):
        valid = (v >= -1.0) & (v <= size)
        vc = jnp.maximum(v, 0.0)
        lo = jnp.minimum(jnp.floor(vc).astype(jnp.int32), size - 1)
        hi = jnp.minimum(lo + 1, size - 1)
        vv = jnp.where(lo >= size - 1, jnp.float32(size - 1), vc)
        l = vv - lo.astype(jnp.float32)
        m = valid.astype(jnp.float32)
        return lo, hi, (1.0 - l) * m, l * m

    iota_j = jax.lax.broadcasted_iota(jnp.int32, (_POOL, _W), 0)
    iota_w = jax.lax.broadcasted_iota(jnp.int32, (_POOL, _W), 1)
    xw = jnp.zeros((_POOL, _W), dtype=jnp.float32)
    for j in range(_POOL):
        for t in range(_SR):
            x = rsw + (j + (t + 0.5) / _SR) * bin_w
            lo, hi, wlo, whi = interp(x, _W)
            sel_j = iota_j == j
            xw = xw + jnp.where(sel_j & (iota_w == lo), wlo * 0.25, 0.0)
            xw = xw + jnp.where(sel_j & (iota_w == hi), whi * 0.25, 0.0)
    xw = xw.astype(jnp.bfloat16)

    for i in range(_POOL):
        acc = None
        for s in range(_SR):
            y = rsh + (i + (s + 0.5) / _SR) * bin_h
            lo, hi, wlo, whi = interp(y, _H)
            r_lo = f_ref[b, lo, :, :]
            r_hi = f_ref[b, hi, :, :]
            term = (wlo.astype(jnp.bfloat16) * r_lo
                    + whi.astype(jnp.bfloat16) * r_hi)
            acc = term if acc is None else acc + term
        res = jax.lax.dot_general(
            xw, acc, (((1,), (0,)), ((), ())),
            preferred_element_type=jnp.float32)
        o_ref[k, i] = res


def _roi_kernel(rois_ref, f_ref, o_ref):
    n0 = pl.program_id(1) * _NR
    for k in range(_NR):
        _one_roi(rois_ref, f_ref, o_ref, n0 + k, k)


def _run(ft, rois, cb):
    n_rois = rois.shape[0]
    cl = ft.shape[3]
    nc = cl // cb
    grid = (nc, n_rois // _NR)
    return pl.pallas_call(
        _roi_kernel,
        grid=grid,
        in_specs=[
            pl.BlockSpec(memory_space=pltpu.SMEM),
            pl.BlockSpec((2, _H, _W, cb), lambda c, n: (0, 0, 0, c)),
        ],
        out_specs=pl.BlockSpec(
            (_NR, _POOL, _POOL, cb), lambda c, n: (n, 0, 0, c)),
        out_shape=jax.ShapeDtypeStruct((n_rois, _POOL, _POOL, cl), jnp.float32),
    )(rois, ft)


@jax.jit
def kernel(input, rois):
    B, C, L, H, W = input.shape
    ft = input.reshape(B, C * L, H, W).astype(jnp.bfloat16).transpose(
        0, 2, 3, 1)
    out = _run(ft, rois, 2048)
    return out.transpose(0, 3, 1, 2).reshape(rois.shape[0], C, L, _POOL, _POOL)
